# Initial kernel scaffold; baseline (speedup 1.0000x reference)
import jax
import jax.numpy as jnp
from jax.experimental import pallas as pl


def kernel(x, edge_index, W0, b0, W1, b1, W2, b2, W3, b3):
    raise NotImplementedError("write your pallas kernel here")


# speedup vs baseline: 4.2795x; 4.2795x over previous
import functools

import jax
import jax.numpy as jnp
from jax import lax
from jax.experimental import pallas as pl
from jax.experimental.pallas import tpu as pltpu
from jax.experimental.pallas import tpu_sc as plsc

N = 10000
E = 320000
D = 128
NC = 2
NS = 16
NW = NC * NS
K = 128
NP = 10240
RA = NP // NW
DUMMY = RA
AR = RA + 8
NB = 48

_MESH = plsc.VectorSubcoreMesh(
    core_axis_name="c", subcore_axis_name="s", num_cores=NC, num_subcores=NS)


def _worker_id():
    return lax.axis_index("s") * NC + lax.axis_index("c")


def _span(bnd_v, wid):
    vec = bnd_v[pl.ds(wid, 16)]
    lo = vec[0]
    hi = vec[1]
    c0 = lo // K
    c1 = jnp.maximum((hi + K - 1) // K, c0)
    return lo, hi, c0, c1


def _dst_local(didx_v, g, base):
    dvec = didx_v[pl.ds(g * 16, 16)]
    rel = dvec - base
    inr = (rel >= 0) & (rel < RA)
    return jnp.where(inr, rel, DUMMY)


@functools.partial(
    pl.kernel,
    out_type=jax.ShapeDtypeStruct((NP, 16), jnp.float32),
    mesh=_MESH,
    scratch_types=[
        pltpu.VMEM((NB,), jnp.int32),
        pltpu.VMEM((K,), jnp.int32),
        pltpu.VMEM((AR, 16), jnp.float32),
        pltpu.SemaphoreType.DMA,
    ],
)
def _deg_kernel(dst_hbm, bnd_hbm, zeros_hbm, out_hbm, bnd_v, didx_v, acc_v, sem):
    wid = _worker_id()
    base = wid * RA

    pltpu.sync_copy(bnd_hbm, bnd_v)
    pltpu.async_copy(zeros_hbm, acc_v, sem).wait()
    lo, hi, c0, c1 = _span(bnd_v, wid)

    onehot = jnp.where(jnp.arange(16) == 0, 1.0, 0.0).astype(jnp.float32)

    def body(c, _):
        pltpu.sync_copy(dst_hbm.at[pl.ds(c * K, K)], didx_v)
        def group(g, _):
            dl = _dst_local(didx_v, g, base)
            for l in range(16):
                r = dl[l]
                plsc.addupdate(acc_v.at[r, pl.ds(0, 16)], onehot)
            return 0
        lax.fori_loop(0, K // 16, group, 0)
        return 0
    lax.fori_loop(c0, c1, body, 0)

    pltpu.sync_copy(acc_v.at[pl.ds(0, RA)], out_hbm.at[pl.ds(base, RA)])


@functools.partial(
    pl.kernel,
    out_type=jax.ShapeDtypeStruct((NP, D), jnp.float32),
    mesh=_MESH,
    scratch_types=[
        pltpu.VMEM((NB,), jnp.int32),
        pltpu.VMEM((K,), jnp.int32),
        pltpu.VMEM((K,), jnp.int32),
        pltpu.VMEM((K, D), jnp.float32),
        pltpu.VMEM((AR, D), jnp.float32),
        pltpu.SemaphoreType.DMA,
    ],
)
def _msgpass_kernel(g_hbm, src_hbm, dst_hbm, bnd_hbm, zeros_hbm, out_hbm,
                    bnd_v, sidx_v, didx_v, rows_v, acc_v, sem):
    wid = _worker_id()
    base = wid * RA

    pltpu.sync_copy(bnd_hbm, bnd_v)
    pltpu.async_copy(zeros_hbm, acc_v, sem).wait()
    lo, hi, c0, c1 = _span(bnd_v, wid)

    def body(c, _):
        pltpu.sync_copy(src_hbm.at[pl.ds(c * K, K)], sidx_v)
        pltpu.sync_copy(dst_hbm.at[pl.ds(c * K, K)], didx_v)
        pltpu.async_copy(g_hbm.at[sidx_v], rows_v, sem).wait()
        def group(g, _):
            dl = _dst_local(didx_v, g, base)
            for l in range(16):
                r = dl[l]
                e = g * 16 + l
                for j in range(D // 16):
                    plsc.addupdate(acc_v.at[r, pl.ds(16 * j, 16)],
                                   rows_v[e, pl.ds(16 * j, 16)])
            return 0
        lax.fori_loop(0, K // 16, group, 0)
        return 0
    lax.fori_loop(c0, c1, body, 0)

    pltpu.sync_copy(acc_v.at[pl.ds(0, RA)], out_hbm.at[pl.ds(base, RA)])


_R = 400
_NB = N // _R


def _layer0_body(x_ref, d_ref, w_ref, g_ref, dv_ref):
    deg = d_ref[:, :1] + 1.0
    dv = lax.rsqrt(deg)
    dv_ref[...] = dv
    g_ref[...] = jnp.dot(x_ref[...], w_ref[...],
                         preferred_element_type=jnp.float32) * dv


_layer0 = pl.pallas_call(
    _layer0_body,
    grid=(_NB,),
    in_specs=[
        pl.BlockSpec((_R, D), lambda i: (i, 0)),
        pl.BlockSpec((_R, 16), lambda i: (i, 0)),
        pl.BlockSpec((D, D), lambda i: (0, 0)),
    ],
    out_specs=[
        pl.BlockSpec((_R, D), lambda i: (i, 0)),
        pl.BlockSpec((_R, 1), lambda i: (i, 0)),
    ],
    out_shape=[
        jax.ShapeDtypeStruct((N, D), jnp.float32),
        jax.ShapeDtypeStruct((N, 1), jnp.float32),
    ],
)


def _mid_body(s_ref, dv_ref, b_ref, w_ref, g_ref):
    dv = dv_ref[...]
    h = jnp.maximum(dv * s_ref[...] + b_ref[...], 0.0)
    g_ref[...] = jnp.dot(h, w_ref[...],
                         preferred_element_type=jnp.float32) * dv


_mid = pl.pallas_call(
    _mid_body,
    grid=(_NB,),
    in_specs=[
        pl.BlockSpec((_R, D), lambda i: (i, 0)),
        pl.BlockSpec((_R, 1), lambda i: (i, 0)),
        pl.BlockSpec((1, D), lambda i: (0, 0)),
        pl.BlockSpec((D, D), lambda i: (0, 0)),
    ],
    out_specs=pl.BlockSpec((_R, D), lambda i: (i, 0)),
    out_shape=jax.ShapeDtypeStruct((N, D), jnp.float32),
)


def _final_body(s_ref, dv_ref, b_ref, o_ref):
    o_ref[...] = dv_ref[...] * s_ref[...] + b_ref[...]


_final = pl.pallas_call(
    _final_body,
    grid=(_NB,),
    in_specs=[
        pl.BlockSpec((_R, D), lambda i: (i, 0)),
        pl.BlockSpec((_R, 1), lambda i: (i, 0)),
        pl.BlockSpec((1, D), lambda i: (0, 0)),
    ],
    out_specs=pl.BlockSpec((_R, D), lambda i: (i, 0)),
    out_shape=jax.ShapeDtypeStruct((N, D), jnp.float32),
)


def kernel(x, edge_index, W0, b0, W1, b1, W2, b2, W3, b3):
    src = edge_index[0]
    dst = edge_index[1]

    order = jnp.argsort(dst)
    srcs = src[order]
    dsts = dst[order]
    bnd = jnp.searchsorted(
        dsts, (jnp.arange(NW + 1, dtype=jnp.int32) * RA).astype(jnp.int32)
    ).astype(jnp.int32)
    bnd = jnp.pad(bnd, (0, NB - (NW + 1)), constant_values=E)

    zeros16 = jnp.zeros((AR, 16), jnp.float32)
    zerosD = jnp.zeros((AR, D), jnp.float32)

    dpad = _deg_kernel(dsts, bnd, zeros16)
    g, dinv = _layer0(x, dpad, W0)
    s = _msgpass_kernel(g, srcs, dsts, bnd, zerosD)
    g = _mid(s, dinv, b0.reshape(1, D), W1)
    s = _msgpass_kernel(g, srcs, dsts, bnd, zerosD)
    g = _mid(s, dinv, b1.reshape(1, D), W2)
    s = _msgpass_kernel(g, srcs, dsts, bnd, zerosD)
    g = _mid(s, dinv, b2.reshape(1, D), W3)
    s = _msgpass_kernel(g, srcs, dsts, bnd, zerosD)
    return _final(s, dinv, b3.reshape(1, D))


# speedup vs baseline: 5.1197x; 1.1963x over previous
import functools

import jax
import jax.numpy as jnp
from jax import lax
from jax.experimental import pallas as pl
from jax.experimental.pallas import tpu as pltpu
from jax.experimental.pallas import tpu_sc as plsc

N = 10000
E = 320000
D = 128
NC = 2
NS = 16
NW = NC * NS
K = 128
NP = 10240
RA = NP // NW
DUMMY = RA
AR = RA + 8
NB = 48

_MESH = plsc.VectorSubcoreMesh(
    core_axis_name="c", subcore_axis_name="s", num_cores=NC, num_subcores=NS)


def _worker_id():
    return lax.axis_index("s") * NC + lax.axis_index("c")


def _span(bnd_v, wid):
    vec = bnd_v[pl.ds(wid, 16)]
    lo = vec[0]
    hi = vec[1]
    c0 = lo // K
    c1 = jnp.maximum((hi + K - 1) // K, c0)
    return lo, hi, c0, c1


def _dst_local(didx_v, g, base):
    dvec = didx_v[pl.ds(g * 16, 16)]
    rel = dvec - base
    inr = (rel >= 0) & (rel < RA)
    return jnp.where(inr, rel, DUMMY)


@functools.partial(
    pl.kernel,
    out_type=jax.ShapeDtypeStruct((NP, 16), jnp.float32),
    mesh=_MESH,
    scratch_types=[
        pltpu.VMEM((NB,), jnp.int32),
        pltpu.VMEM((K,), jnp.int32),
        pltpu.VMEM((AR, 16), jnp.float32),
        pltpu.SemaphoreType.DMA,
    ],
)
def _deg_kernel(dst_hbm, bnd_hbm, zeros_hbm, out_hbm, bnd_v, didx_v, acc_v, sem):
    wid = _worker_id()
    base = wid * RA

    pltpu.sync_copy(bnd_hbm, bnd_v)
    pltpu.async_copy(zeros_hbm, acc_v, sem).wait()
    lo, hi, c0, c1 = _span(bnd_v, wid)

    onehot = jnp.where(jnp.arange(16) == 0, 1.0, 0.0).astype(jnp.float32)

    def body(c, _):
        pltpu.sync_copy(dst_hbm.at[pl.ds(c * K, K)], didx_v)
        def group(g, _):
            dl = _dst_local(didx_v, g, base)
            for l in range(16):
                r = dl[l]
                plsc.addupdate(acc_v.at[r, pl.ds(0, 16)], onehot)
            return 0
        lax.fori_loop(0, K // 16, group, 0)
        return 0
    lax.fori_loop(c0, c1, body, 0)

    pltpu.sync_copy(acc_v.at[pl.ds(0, RA)], out_hbm.at[pl.ds(base, RA)])


@functools.partial(
    pl.kernel,
    out_type=jax.ShapeDtypeStruct((NP, D), jnp.float32),
    mesh=_MESH,
    scratch_types=[
        pltpu.VMEM((NB,), jnp.int32),
        pltpu.VMEM((2, K), jnp.int32),
        pltpu.VMEM((2, K), jnp.int32),
        pltpu.VMEM((2, K, D), jnp.float32),
        pltpu.VMEM((AR, D), jnp.float32),
        pltpu.SemaphoreType.DMA,
        pltpu.SemaphoreType.DMA((2,)),
    ],
)
def _msgpass_kernel(g_hbm, src_hbm, dst_hbm, bnd_hbm, zeros_hbm, out_hbm,
                    bnd_v, sidx_v, didx_v, rows_v, acc_v, sem, gsems):
    wid = _worker_id()
    base = wid * RA

    pltpu.sync_copy(bnd_hbm, bnd_v)
    pltpu.async_copy(zeros_hbm, acc_v, sem).wait()
    lo, hi, c0, c1 = _span(bnd_v, wid)

    def stage_and_gather(c, p):
        pltpu.sync_copy(src_hbm.at[pl.ds(c * K, K)], sidx_v.at[p])
        pltpu.sync_copy(dst_hbm.at[pl.ds(c * K, K)], didx_v.at[p])
        pltpu.async_copy(g_hbm.at[sidx_v.at[p]], rows_v.at[p], gsems.at[p])

    @pl.when(c0 < c1)
    def _():
        stage_and_gather(c0, 0)

    def body(c, _):
        p = lax.rem(c - c0, 2)
        @pl.when(c + 1 < c1)
        def _():
            stage_and_gather(c + 1, 1 - p)
        pltpu.make_async_copy(g_hbm.at[sidx_v.at[p]],
                              rows_v.at[p], gsems.at[p]).wait()
        def group(g, _):
            dl = _dst_local(didx_v.at[p], g, base)
            for l in range(16):
                r = dl[l]
                e = g * 16 + l
                for j in range(D // 16):
                    plsc.addupdate(acc_v.at[r, pl.ds(16 * j, 16)],
                                   rows_v[p, e, pl.ds(16 * j, 16)])
            return 0
        lax.fori_loop(0, K // 16, group, 0)
        return 0
    lax.fori_loop(c0, c1, body, 0)

    pltpu.sync_copy(acc_v.at[pl.ds(0, RA)], out_hbm.at[pl.ds(base, RA)])


_R = 400
_NB = N // _R


def _layer0_body(x_ref, d_ref, w_ref, g_ref, dv_ref):
    deg = d_ref[:, :1] + 1.0
    dv = lax.rsqrt(deg)
    dv_ref[...] = dv
    g_ref[...] = jnp.dot(x_ref[...], w_ref[...],
                         preferred_element_type=jnp.float32) * dv


_layer0 = pl.pallas_call(
    _layer0_body,
    grid=(_NB,),
    in_specs=[
        pl.BlockSpec((_R, D), lambda i: (i, 0)),
        pl.BlockSpec((_R, 16), lambda i: (i, 0)),
        pl.BlockSpec((D, D), lambda i: (0, 0)),
    ],
    out_specs=[
        pl.BlockSpec((_R, D), lambda i: (i, 0)),
        pl.BlockSpec((_R, 1), lambda i: (i, 0)),
    ],
    out_shape=[
        jax.ShapeDtypeStruct((N, D), jnp.float32),
        jax.ShapeDtypeStruct((N, 1), jnp.float32),
    ],
)


def _mid_body(s_ref, dv_ref, b_ref, w_ref, g_ref):
    dv = dv_ref[...]
    h = jnp.maximum(dv * s_ref[...] + b_ref[...], 0.0)
    g_ref[...] = jnp.dot(h, w_ref[...],
                         preferred_element_type=jnp.float32) * dv


_mid = pl.pallas_call(
    _mid_body,
    grid=(_NB,),
    in_specs=[
        pl.BlockSpec((_R, D), lambda i: (i, 0)),
        pl.BlockSpec((_R, 1), lambda i: (i, 0)),
        pl.BlockSpec((1, D), lambda i: (0, 0)),
        pl.BlockSpec((D, D), lambda i: (0, 0)),
    ],
    out_specs=pl.BlockSpec((_R, D), lambda i: (i, 0)),
    out_shape=jax.ShapeDtypeStruct((N, D), jnp.float32),
)


def _final_body(s_ref, dv_ref, b_ref, o_ref):
    o_ref[...] = dv_ref[...] * s_ref[...] + b_ref[...]


_final = pl.pallas_call(
    _final_body,
    grid=(_NB,),
    in_specs=[
        pl.BlockSpec((_R, D), lambda i: (i, 0)),
        pl.BlockSpec((_R, 1), lambda i: (i, 0)),
        pl.BlockSpec((1, D), lambda i: (0, 0)),
    ],
    out_specs=pl.BlockSpec((_R, D), lambda i: (i, 0)),
    out_shape=jax.ShapeDtypeStruct((N, D), jnp.float32),
)


def kernel(x, edge_index, W0, b0, W1, b1, W2, b2, W3, b3):
    src = edge_index[0]
    dst = edge_index[1]

    order = jnp.argsort(dst)
    srcs = src[order]
    dsts = dst[order]
    bnd = jnp.searchsorted(
        dsts, (jnp.arange(NW + 1, dtype=jnp.int32) * RA).astype(jnp.int32)
    ).astype(jnp.int32)
    bnd = jnp.pad(bnd, (0, NB - (NW + 1)), constant_values=E)

    zeros16 = jnp.zeros((AR, 16), jnp.float32)
    zerosD = jnp.zeros((AR, D), jnp.float32)

    dpad = _deg_kernel(dsts, bnd, zeros16)
    g, dinv = _layer0(x, dpad, W0)
    s = _msgpass_kernel(g, srcs, dsts, bnd, zerosD)
    g = _mid(s, dinv, b0.reshape(1, D), W1)
    s = _msgpass_kernel(g, srcs, dsts, bnd, zerosD)
    g = _mid(s, dinv, b1.reshape(1, D), W2)
    s = _msgpass_kernel(g, srcs, dsts, bnd, zerosD)
    g = _mid(s, dinv, b2.reshape(1, D), W3)
    s = _msgpass_kernel(g, srcs, dsts, bnd, zerosD)
    return _final(s, dinv, b3.reshape(1, D))
